# baseline (speedup 1.0000x reference)
import functools

import jax
import jax.numpy as jnp
from jax import lax
from jax.experimental import pallas as pl
from jax.experimental.pallas import tpu as pltpu
from jax.experimental.pallas import tpu_sc as plsc

SPARSITY = 0.1
L = 16
NC = 2
NS = 16
NW = NC * NS
CHUNK = 4
NBUF = 4
U = 8
SLOT_CAP = 64


def _row_select(rbuf, hist, hsbuf, hist16, cbuf, rb, n, k):
    nv = n // L
    lanes = lax.iota(jnp.int32, L)
    bankA = lanes * 256
    ones_i = jnp.ones((L,), jnp.int32)
    zeros_i = jnp.zeros((L,), jnp.int32)
    scale = jnp.float32(n / k)
    kmask = jnp.int32(0x7FFFFFFF)

    def keys_at(off):
        v = rbuf[pl.ds(off, L)]
        return lax.bitcast_convert_type(v, jnp.int32) & kmask, v

    @plsc.parallel_loop(0, nv, unroll=U)
    def _scan0(i):
        kv, _ = keys_at(rb + i * L)
        plsc.addupdate_scatter(hist, [(kv >> 23) + bankA], ones_i)

    @plsc.parallel_loop(0, 16, unroll=2)
    def _red0(v):
        acc = zeros_i
        for lane in range(16):
            sl = pl.ds(lane * 256 + v * L, L)
            acc = acc + hist[sl]
            hist[sl] = zeros_i
        hsbuf[pl.ds(v * L, L)] = acc

    k_rem = jnp.int32(k)

    @plsc.parallel_loop(0, 16, carry=(jnp.int32(0), jnp.int32(0),
                                      jnp.int32(0)))
    def _sel0(j, carry):
        running, nq, ca = carry
        cv = hsbuf[pl.ds((15 - j) * L, L)]
        rc = plsc.cumsum(jnp.flip(cv, axis=0))
        rcq = rc + running
        qual = rcq >= k_rem
        nq = nq + jnp.sum(qual.astype(jnp.int32))
        ca = jnp.maximum(ca, jnp.max(jnp.where(qual, 0, rcq)))
        running = running + jnp.max(rc)
        return running, nq, ca
    _, nq0, ca0 = _sel0
    p = nq0 - 1
    k_rem = k_rem - ca0

    @plsc.parallel_loop(0, nv, unroll=U, carry=zeros_i)
    def _comp(i, percount):
        kv, _ = keys_at(rb + i * L)
        pm = (kv >> 23) == p
        plsc.store_scatter(cbuf, [(percount << 4) + lanes], kv, mask=pm)
        return percount + pm.astype(jnp.int32)
    percount = _comp
    mmax = jnp.max(percount)

    def sel16(kr):
        @plsc.parallel_loop(0, 16, carry=zeros_i)
        def _t(j, acc):
            sl = pl.ds(j * L, L)
            t = jnp.sum(hist16[sl])
            hist16[sl] = zeros_i
            return jnp.where(lanes == j, t, acc)
        rc = plsc.cumsum(jnp.flip(_t, axis=0))
        qual = rc >= kr
        nq = jnp.sum(qual.astype(jnp.int32))
        ca = jnp.max(jnp.where(qual, 0, rc))
        return nq - 1, ca

    def level_scan(shift, p, final):
        def body(j):
            kv = cbuf[pl.ds(j * L, L)]
            if final:
                pm = ((kv >> 3) == p) & (j < percount)
                binv = kv & 15
            else:
                pm = ((kv >> (shift + 4)) == p) & (j < percount)
                binv = (kv >> shift) & 15
            plsc.addupdate_scatter(hist16, [(binv << 4) + lanes],
                                   ones_i, mask=pm)

        @plsc.parallel_loop(0, SLOT_CAP, unroll=4)
        def _fast(j):
            body(j)

        def _slow(j, c):
            body(j)
            return c
        lax.fori_loop(SLOT_CAP, jnp.maximum(mmax, SLOT_CAP), _slow, 0)

    for shift in (19, 15, 11, 7, 3):
        level_scan(shift, p, final=False)
        b, ca = sel16(k_rem)
        p = (p << 4) | b
        k_rem = k_rem - ca

    level_scan(0, p, final=True)
    b6, _ = sel16(k_rem)
    thr = (p << 3) | (b6 & 7)

    @plsc.parallel_loop(0, nv, unroll=U)
    def _outb(i):
        off = rb + i * L
        kv, v = keys_at(off)
        rbuf[pl.ds(off, L)] = jnp.where(kv >= thr, v * scale,
                                        jnp.float32(0.0))


def _make_sc_kernel(rows, n, k):
    rpw = rows // NW
    nchunk = rpw // CHUNK
    assert nchunk % NBUF == 0 and nchunk >= 2 * NBUF
    cn = CHUNK * n
    mesh = plsc.VectorSubcoreMesh(core_axis_name="c", subcore_axis_name="s",
                                  num_cores=NC, num_subcores=NS)

    @functools.partial(
        pl.kernel,
        out_type=jax.ShapeDtypeStruct((rows * n,), jnp.float32),
        mesh=mesh,
        compiler_params=pltpu.CompilerParams(needs_layout_passes=False),
        scratch_types=[
            [pltpu.VMEM((cn,), jnp.float32) for _ in range(NBUF)],
            pltpu.VMEM((16 * 256,), jnp.int32),
            pltpu.VMEM((256,), jnp.int32),
            pltpu.VMEM((256,), jnp.int32),
            pltpu.VMEM((16 * 256,), jnp.int32),
            [pltpu.SemaphoreType.DMA for _ in range(NBUF)],
            [pltpu.SemaphoreType.DMA for _ in range(NBUF)],
        ],
    )
    def sc_kernel(x_hbm, o_hbm, bufs, hist, hsbuf, hist16, cbuf, sin, sout):
        cid = lax.axis_index("c")
        sid = lax.axis_index("s")
        wid = sid * NC + cid
        base0 = wid * rpw * n
        zeros_i = jnp.zeros((L,), jnp.int32)

        @plsc.parallel_loop(0, 256, unroll=4)
        def _z(i):
            hist[pl.ds(i * L, L)] = zeros_i

        @plsc.parallel_loop(0, 16)
        def _z16(i):
            hist16[pl.ds(i * L, L)] = zeros_i

        def in_copy(c, b):
            return pltpu.make_async_copy(
                x_hbm.at[pl.ds(base0 + c * cn, cn)], bufs[b], sin[b])

        def out_copy(c, b):
            return pltpu.make_async_copy(
                bufs[b], o_hbm.at[pl.ds(base0 + c * cn, cn)], sout[b])

        for b in range(NBUF - 1):
            in_copy(jnp.int32(b), b).start()

        def grp(g, _):
            for b in range(NBUF):
                c = g * NBUF + b
                in_copy(c, b).wait()

                def rowloop(r, _):
                    _row_select(bufs[b], hist, hsbuf, hist16, cbuf,
                                r * n, n, k)
                    return 0
                lax.fori_loop(0, CHUNK, rowloop, 0)
                out_copy(c, b).start()

                nxt = c + NBUF - 1
                tb = (b + NBUF - 1) % NBUF

                @pl.when(nxt < nchunk)
                def _():
                    @pl.when(c > 0)
                    def _():
                        out_copy(c - 1, tb).wait()
                    in_copy(nxt, tb).start()
            return 0
        lax.fori_loop(0, nchunk // NBUF, grp, 0)

        for b in range(NBUF):
            out_copy(jnp.int32(nchunk - NBUF + b), (nchunk - NBUF + b) % NBUF).wait()

    return sc_kernel


def kernel(x):
    b, s, n = x.shape
    k = max(1, int(n * SPARSITY))
    rows = b * s
    out = _make_sc_kernel(rows, n, k)(x.reshape(rows * n))
    return out.reshape(b, s, n)


# speedup vs baseline: 1.1697x; 1.1697x over previous
import functools

import jax
import jax.numpy as jnp
from jax import lax
from jax.experimental import pallas as pl
from jax.experimental.pallas import tpu as pltpu
from jax.experimental.pallas import tpu_sc as plsc

SPARSITY = 0.1
L = 16
NC = 2
NS = 16
NW = NC * NS
CHUNK = 4
NBUF = 4
U = 8
SLOT_CAP = 96


def _row_select(rbuf, hist, hsbuf, hist16, cbuf, rb, n, k):
    nv = n // L
    lanes = lax.iota(jnp.int32, L)
    bankA = lanes * 256
    ones_i = jnp.ones((L,), jnp.int32)
    zeros_i = jnp.zeros((L,), jnp.int32)
    scale = jnp.float32(n / k)
    kmask = jnp.int32(0x7FFFFFFF)

    def keys_at(off):
        v = rbuf[pl.ds(off, L)]
        return lax.bitcast_convert_type(v, jnp.int32) & kmask, v

    @plsc.parallel_loop(0, nv, unroll=U)
    def _scan0(i):
        kv, _ = keys_at(rb + i * L)
        plsc.addupdate_scatter(hist, [(kv >> 23) + bankA], ones_i)

    @plsc.parallel_loop(0, 16, unroll=2)
    def _red0(v):
        acc = zeros_i
        for lane in range(16):
            sl = pl.ds(lane * 256 + v * L, L)
            acc = acc + hist[sl]
            hist[sl] = zeros_i
        hsbuf[pl.ds(v * L, L)] = acc

    k_rem = jnp.int32(k)

    @plsc.parallel_loop(0, 16, carry=(jnp.int32(0), jnp.int32(0),
                                      jnp.int32(0)))
    def _sel0(j, carry):
        running, nq, ca = carry
        cv = hsbuf[pl.ds((15 - j) * L, L)]
        rc = plsc.cumsum(jnp.flip(cv, axis=0))
        rcq = rc + running
        qual = rcq >= k_rem
        nq = nq + jnp.sum(qual.astype(jnp.int32))
        ca = jnp.maximum(ca, jnp.max(jnp.where(qual, 0, rcq)))
        running = running + jnp.max(rc)
        return running, nq, ca
    _, nq0, ca0 = _sel0
    p = nq0 - 1
    k_rem = k_rem - ca0

    @plsc.parallel_loop(0, nv, unroll=U, carry=zeros_i)
    def _comp(i, percount):
        kv, _ = keys_at(rb + i * L)
        pm = (kv >> 23) == p
        plsc.store_scatter(cbuf, [(percount << 4) + lanes], kv, mask=pm)
        return percount + pm.astype(jnp.int32)
    percount = _comp
    mmax = jnp.max(percount)

    def sel16(kr):
        @plsc.parallel_loop(0, 16, carry=(jnp.int32(0), jnp.int32(0),
                                          jnp.int32(0)))
        def _s(b, carry):
            running, nq, ca = carry
            binv = 15 - b
            sl = pl.ds(binv * L, L)
            cvec = hist16[sl]
            hist16[sl] = zeros_i
            running = running + jnp.sum(cvec)
            qual = running >= kr
            nq = nq + qual.astype(jnp.int32)
            ca = jnp.where(running < kr, running, ca)
            return running, nq, ca
        _, nq, ca = _s
        return nq - 1, ca

    def level_scan(shift, p, final):
        def body(j):
            kv = cbuf[pl.ds(j * L, L)]
            if final:
                pm = ((kv >> 3) == p) & (j < percount)
                binv = kv & 15
            else:
                pm = ((kv >> (shift + 4)) == p) & (j < percount)
                binv = (kv >> shift) & 15
            plsc.addupdate_scatter(hist16, [(binv << 4) + lanes],
                                   ones_i, mask=pm)

        @plsc.parallel_loop(0, SLOT_CAP, unroll=4)
        def _fast(j):
            body(j)

        def _slow(j, c):
            body(j)
            return c
        lax.fori_loop(SLOT_CAP, jnp.maximum(mmax, SLOT_CAP), _slow, 0)

    for shift in (19, 15, 11, 7, 3):
        level_scan(shift, p, final=False)
        b, ca = sel16(k_rem)
        p = (p << 4) | b
        k_rem = k_rem - ca

    level_scan(0, p, final=True)
    b6, _ = sel16(k_rem)
    thr = (p << 3) | (b6 & 7)

    @plsc.parallel_loop(0, nv, unroll=U)
    def _outb(i):
        off = rb + i * L
        kv, v = keys_at(off)
        rbuf[pl.ds(off, L)] = jnp.where(kv >= thr, v * scale,
                                        jnp.float32(0.0))


def _make_sc_kernel(rows, n, k):
    rpw = rows // NW
    nchunk = rpw // CHUNK
    assert nchunk % NBUF == 0 and nchunk >= 2 * NBUF
    cn = CHUNK * n
    mesh = plsc.VectorSubcoreMesh(core_axis_name="c", subcore_axis_name="s",
                                  num_cores=NC, num_subcores=NS)

    @functools.partial(
        pl.kernel,
        out_type=jax.ShapeDtypeStruct((rows * n,), jnp.float32),
        mesh=mesh,
        compiler_params=pltpu.CompilerParams(needs_layout_passes=False),
        scratch_types=[
            [pltpu.VMEM((cn,), jnp.float32) for _ in range(NBUF)],
            pltpu.VMEM((16 * 256,), jnp.int32),
            pltpu.VMEM((256,), jnp.int32),
            pltpu.VMEM((256,), jnp.int32),
            pltpu.VMEM((16 * 256,), jnp.int32),
            [pltpu.SemaphoreType.DMA for _ in range(NBUF)],
            [pltpu.SemaphoreType.DMA for _ in range(NBUF)],
        ],
    )
    def sc_kernel(x_hbm, o_hbm, bufs, hist, hsbuf, hist16, cbuf, sin, sout):
        cid = lax.axis_index("c")
        sid = lax.axis_index("s")
        wid = sid * NC + cid
        base0 = wid * rpw * n
        zeros_i = jnp.zeros((L,), jnp.int32)

        @plsc.parallel_loop(0, 256, unroll=4)
        def _z(i):
            hist[pl.ds(i * L, L)] = zeros_i

        @plsc.parallel_loop(0, 16)
        def _z16(i):
            hist16[pl.ds(i * L, L)] = zeros_i

        def in_copy(c, b):
            return pltpu.make_async_copy(
                x_hbm.at[pl.ds(base0 + c * cn, cn)], bufs[b], sin[b])

        def out_copy(c, b):
            return pltpu.make_async_copy(
                bufs[b], o_hbm.at[pl.ds(base0 + c * cn, cn)], sout[b])

        for b in range(NBUF - 1):
            in_copy(jnp.int32(b), b).start()

        def grp(g, _):
            for b in range(NBUF):
                c = g * NBUF + b
                in_copy(c, b).wait()

                def rowloop(r, _):
                    _row_select(bufs[b], hist, hsbuf, hist16, cbuf,
                                r * n, n, k)
                    return 0
                lax.fori_loop(0, CHUNK, rowloop, 0)
                out_copy(c, b).start()

                nxt = c + NBUF - 1
                tb = (b + NBUF - 1) % NBUF

                @pl.when(nxt < nchunk)
                def _():
                    @pl.when(c > 0)
                    def _():
                        out_copy(c - 1, tb).wait()
                    in_copy(nxt, tb).start()
            return 0
        lax.fori_loop(0, nchunk // NBUF, grp, 0)

        for b in range(NBUF):
            out_copy(jnp.int32(nchunk - NBUF + b), (nchunk - NBUF + b) % NBUF).wait()

    return sc_kernel


def kernel(x):
    b, s, n = x.shape
    k = max(1, int(n * SPARSITY))
    rows = b * s
    out = _make_sc_kernel(rows, n, k)(x.reshape(rows * n))
    return out.reshape(b, s, n)
